# Initial kernel scaffold; baseline (speedup 1.0000x reference)
import jax
import jax.numpy as jnp
from jax.experimental import pallas as pl


def kernel(Q, K, V, Wq, bq, Wk, bk, Wv, bv, Wo, bo):
    raise NotImplementedError("write your pallas kernel here")


# speedup vs baseline: 1.5893x; 1.5893x over previous
import numpy as np
import jax
import jax.numpy as jnp
from jax.experimental import pallas as pl
from jax.experimental.pallas import tpu as pltpu

_B, _L, _D = 4, 4096, 768
_NH, _DK = 12, 64
_N1 = _N2 = 64
_NJ = 12
_JB = 64
_KT = 8
_RB = 512
_LB = 256

_ar = np.arange(64)
_F64 = np.exp(-2j * np.pi * np.outer(_ar, _ar) / 64.0)
_TW = np.exp(-2j * np.pi * np.outer(_ar, _ar) / 4096.0)
_TWB = np.tile(_TW, (1, _JB))
_F_RE = np.ascontiguousarray(_F64.real.astype(np.float32))
_F_IM = np.ascontiguousarray(_F64.imag.astype(np.float32))
_TB_RE = np.ascontiguousarray(_TWB.real.astype(np.float32))
_TB_IM = np.ascontiguousarray(_TWB.imag.astype(np.float32))
_JONES = np.tile(np.eye(64, dtype=np.float32), (_JB, 1))


def _proj_body(q_ref, k_ref, v_ref, wq_ref, wk_ref, wv_ref, wo_ref, b_ref,
               qo_ref, ko_ref, vo_ref):
    bq = b_ref[0, :]
    bk = b_ref[1, :]
    bv = b_ref[2, :]
    bo = b_ref[3, :]
    dot = lambda a, b: jnp.dot(a, b, preferred_element_type=jnp.float32)
    qo_ref[...] = dot(q_ref[...], wq_ref[...]) + bq[None, :]
    ko_ref[...] = dot(k_ref[...], wk_ref[...]) + bk[None, :]
    v = dot(v_ref[...], wv_ref[...]) + bv[None, :]
    vo_ref[...] = dot(v, wo_ref[...]) + bo[None, :]


def _f1_body(q_ref, k_ref, fre_ref, fim_ref, tre_ref, tim_ref,
             qcr_ref, qci_ref, kcr_ref, kci_ref):
    fre = fre_ref[...]
    fim = fim_ref[...]
    tre = tre_ref[...]
    tim = tim_ref[...]
    dot = lambda a, b: jnp.dot(a, b, preferred_element_type=jnp.float32)

    def stage1(x):
        br = dot(fre, x)
        bi = dot(fim, x)
        return br * tre - bi * tim, br * tim + bi * tre

    qcr, qci = stage1(q_ref[0, 0])
    qcr_ref[0, 0] = qcr
    qci_ref[0, 0] = qci
    kcr, kci = stage1(k_ref[0, 0])
    kcr_ref[0, 0] = kcr
    kci_ref[0, 0] = kci


def _f2_body(qcr_ref, qci_ref, kcr_ref, kci_ref, fre_ref, fim_ref,
             er_ref, ei_ref):
    fre = fre_ref[...]
    fim = fim_ref[...]
    dot = lambda a, b: jnp.dot(a, b, preferred_element_type=jnp.float32)

    def stage2(cr, ci):
        return dot(cr, fre) - dot(ci, fim), dot(cr, fim) + dot(ci, fre)

    qr, qi = stage2(qcr_ref[0, 0], qci_ref[0, 0])
    kr, ki = stage2(kcr_ref[0, 0], kci_ref[0, 0])
    sr = qr * kr + qi * ki
    si = qi * kr - qr * ki
    er_ref[0, 0] = dot(sr, fre) + dot(si, fim)
    ei_ref[0, 0] = dot(si, fre) - dot(sr, fim)


def _f3_body(er_ref, ei_ref, fre_ref, fim_ref, tre_ref, tim_ref, jones_ref,
             corr_ref, r_ref):
    jb = pl.program_id(1)
    fre = fre_ref[...]
    fim = fim_ref[...]
    tre = tre_ref[...]
    tim = tim_ref[...]
    dot = lambda a, b: jnp.dot(a, b, preferred_element_type=jnp.float32)
    er = er_ref[0, 0]
    ei = ei_ref[0, 0]
    cr = er * tre + ei * tim
    ci = ei * tre - er * tim
    ar = (dot(fre, cr) + dot(fim, ci)) * (1.0 / 4096.0)
    corr_ref[0, 0] = ar
    rblk = dot(ar, jones_ref[...])

    @pl.when(jb == 0)
    def _():
        r_ref[0] = rblk

    @pl.when(jb != 0)
    def _():
        r_ref[0] = r_ref[0] + rblk


def _topk_body(r_ref, tau_ref, wgt_ref):
    rm = r_ref[0] + r_ref[1] + r_ref[2] + r_ref[3]
    row = jax.lax.broadcasted_iota(jnp.int32, (64, 64), 0)
    col = jax.lax.broadcasted_iota(jnp.int32, (64, 64), 1)
    lin = row * 64 + col
    big = jnp.int32(1 << 30)
    neg = jnp.float32(-3.0e38)
    tr = [[None] * _KT for _ in range(_B)]
    for i in range(_KT):
        m = jnp.max(rm)
        idx = jnp.min(jnp.where(rm == m, lin, big))
        tau_ref[i] = idx
        sel = lin == idx
        for b in range(_B):
            tr[b][i] = jnp.sum(jnp.where(sel, r_ref[b], 0.0)) * (1.0 / 768.0)
        rm = jnp.where(sel, neg, rm)
    for b in range(_B):
        mx = tr[b][0]
        for i in range(1, _KT):
            mx = jnp.maximum(mx, tr[b][i])
        es = [jnp.exp(tr[b][i] - mx) for i in range(_KT)]
        tot = es[0]
        for i in range(1, _KT):
            tot = tot + es[i]
        inv = 1.0 / tot
        for i in range(_KT):
            wgt_ref[b, i] = es[i] * inv


def _agg_body(tau_ref, wgt_ref, vo_ref, out_ref, *scratch):
    bufs = scratch[:_KT]
    sems = scratch[_KT]
    b = pl.program_id(0)
    l0 = pl.program_id(1) * _LB
    copies = []
    rems = []
    for i in range(_KT):
        start = l0 + tau_ref[i]
        base = (start // 8) * 8
        rems.append(start - base)
        cp = pltpu.make_async_copy(
            vo_ref.at[b, pl.ds(base, _LB + 8), :], bufs[i], sems.at[i])
        cp.start()
        copies.append(cp)
    acc = None
    for i in range(_KT):
        copies[i].wait()
        x = bufs[i][...]
        w = wgt_ref[b, i]
        for j in range(8):
            c = jnp.where(rems[i] == j, w, 0.0)
            term = c * x[j:j + _LB, :]
            acc = term if acc is None else acc + term
    out_ref[0] = acc


def kernel(Q, K, V, Wq, bq, Wk, bk, Wv, bv, Wo, bo):
    f32 = jnp.float32
    fre = jnp.asarray(_F_RE)
    fim = jnp.asarray(_F_IM)
    tre = jnp.asarray(_TB_RE)
    tim = jnp.asarray(_TB_IM)
    jones = jnp.asarray(_JONES)

    q2 = Q.reshape(_B * _L, _D)
    k2 = K.reshape(_B * _L, _D)
    v2 = V.reshape(_B * _L, _D)
    wqt = Wq.T
    wkt = Wk.T
    wvt = Wv.T
    wot = Wo.T
    bias = jnp.stack([bq, bk, bv, bo])

    nrb = (_B * _L) // _RB
    row_spec = pl.BlockSpec((_RB, _D), lambda i: (i, 0))
    w_spec = pl.BlockSpec((_D, _D), lambda i: (0, 0))
    b_spec = pl.BlockSpec((4, _D), lambda i: (0, 0))
    qf, kf, vof = pl.pallas_call(
        _proj_body,
        grid=(nrb,),
        in_specs=[row_spec, row_spec, row_spec, w_spec, w_spec, w_spec,
                  w_spec, b_spec],
        out_specs=[row_spec, row_spec, row_spec],
        out_shape=[jax.ShapeDtypeStruct((_B * _L, _D), f32)] * 3,
    )(q2, k2, v2, wqt, wkt, wvt, wot, bias)

    def to_fft_layout(x):
        x4 = x.reshape(_B, _N1, _N2, _NJ, _JB)
        return x4.transpose(0, 3, 1, 4, 2).reshape(_B, _NJ, 64, _JB * 64)

    qA = to_fft_layout(qf)
    kA = to_fft_layout(kf)

    blk_spec = pl.BlockSpec((1, 1, 64, _JB * 64), lambda b, j: (b, j, 0, 0))
    c_spec = pl.BlockSpec((64, 64), lambda b, j: (0, 0))
    t_spec = pl.BlockSpec((64, _JB * 64), lambda b, j: (0, 0))
    fshape = jax.ShapeDtypeStruct((_B, _NJ, 64, _JB * 64), f32)

    qcr, qci, kcr, kci = pl.pallas_call(
        _f1_body,
        grid=(_B, _NJ),
        in_specs=[blk_spec, blk_spec, c_spec, c_spec, t_spec, t_spec],
        out_specs=[blk_spec] * 4,
        out_shape=[fshape] * 4,
    )(qA, kA, fre, fim, tre, tim)

    rs = lambda x: x.reshape(_B, _NJ, 64 * _JB, 64)
    blk2_spec = pl.BlockSpec((1, 1, 64 * _JB, 64), lambda b, j: (b, j, 0, 0))
    f2shape = jax.ShapeDtypeStruct((_B, _NJ, 64 * _JB, 64), f32)
    er, ei = pl.pallas_call(
        _f2_body,
        grid=(_B, _NJ),
        in_specs=[blk2_spec] * 4 + [c_spec, c_spec],
        out_specs=[blk2_spec] * 2,
        out_shape=[f2shape] * 2,
    )(rs(qcr), rs(qci), rs(kcr), rs(kci), fre, fim)

    jones_spec = pl.BlockSpec((64 * _JB, 64), lambda b, j: (0, 0))
    r_spec = pl.BlockSpec((1, 64, 64), lambda b, j: (b, 0, 0))
    corr5, r_out = pl.pallas_call(
        _f3_body,
        grid=(_B, _NJ),
        in_specs=[blk_spec, blk_spec, c_spec, c_spec, t_spec, t_spec,
                  jones_spec],
        out_specs=[blk_spec, r_spec],
        out_shape=[fshape, jax.ShapeDtypeStruct((_B, 64, 64), f32)],
        compiler_params=pltpu.CompilerParams(
            dimension_semantics=("arbitrary", "arbitrary")),
    )(er.reshape(_B, _NJ, 64, _JB * 64), ei.reshape(_B, _NJ, 64, _JB * 64),
      fre, fim, tre, tim, jones)

    corr_out = (corr5.reshape(_B, _NJ, _N1, _JB, _N2)
                .transpose(0, 2, 4, 1, 3)
                .reshape(_B, _L, _NH, _DK))

    tau, wgt = pl.pallas_call(
        _topk_body,
        grid=(1,),
        in_specs=[pl.BlockSpec((_B, 64, 64), lambda i: (0, 0, 0))],
        out_specs=[pl.BlockSpec(memory_space=pltpu.SMEM),
                   pl.BlockSpec(memory_space=pltpu.SMEM)],
        out_shape=[jax.ShapeDtypeStruct((_KT,), jnp.int32),
                   jax.ShapeDtypeStruct((_B, _KT), f32)],
    )(r_out)

    vo3 = vof.reshape(_B, _L, _D)
    vo_pad = jnp.concatenate([vo3, vo3], axis=1)

    out = pl.pallas_call(
        _agg_body,
        grid=(_B, _L // _LB),
        in_specs=[pl.BlockSpec(memory_space=pltpu.SMEM),
                  pl.BlockSpec(memory_space=pltpu.SMEM),
                  pl.BlockSpec(memory_space=pl.ANY)],
        out_specs=pl.BlockSpec((1, _LB, _D), lambda b, i: (b, i, 0)),
        out_shape=jax.ShapeDtypeStruct((_B, _L, _D), f32),
        scratch_shapes=[pltpu.VMEM((_LB + 8, _D), f32)] * _KT
                       + [pltpu.SemaphoreType.DMA((_KT,))],
    )(tau, wgt, vo_pad)

    return out, corr_out


# speedup vs baseline: 1.8726x; 1.1783x over previous
import numpy as np
import jax
import jax.numpy as jnp
from jax.experimental import pallas as pl
from jax.experimental.pallas import tpu as pltpu

_B, _L, _D = 4, 4096, 768
_NH, _DK = 12, 64
_N1 = _N2 = 64
_NJ = 12
_JB = 64
_KT = 8
_RB = 512
_LB = 256

_ar = np.arange(64)
_F64 = np.exp(-2j * np.pi * np.outer(_ar, _ar) / 64.0)
_TW = np.exp(-2j * np.pi * np.outer(_ar, _ar) / 4096.0)
_TWB = np.tile(_TW, (1, _JB))
_F_RE = np.ascontiguousarray(_F64.real.astype(np.float32))
_F_IM = np.ascontiguousarray(_F64.imag.astype(np.float32))
_TB_RE = np.ascontiguousarray(_TWB.real.astype(np.float32))
_TB_IM = np.ascontiguousarray(_TWB.imag.astype(np.float32))
_JONES = np.tile(np.eye(64, dtype=np.float32), (_JB, 1))


def _proj_body(q_ref, k_ref, v_ref, wq_ref, wk_ref, wv_ref, wo_ref, b_ref,
               qo_ref, ko_ref, vo_ref):
    bq = b_ref[0, :]
    bk = b_ref[1, :]
    bv = b_ref[2, :]
    bo = b_ref[3, :]
    dot = lambda a, b: jnp.dot(a, b, preferred_element_type=jnp.float32)
    qo_ref[...] = dot(q_ref[...], wq_ref[...]) + bq[None, :]
    ko_ref[...] = dot(k_ref[...], wk_ref[...]) + bk[None, :]
    v = dot(v_ref[...], wv_ref[...]) + bv[None, :]
    vo_ref[...] = dot(v, wo_ref[...]) + bo[None, :]


def _f1_body(q_ref, k_ref, fre_ref, fim_ref, tre_ref, tim_ref,
             qcr_ref, qci_ref, kcr_ref, kci_ref):
    fre = fre_ref[...]
    fim = fim_ref[...]
    tre = tre_ref[...]
    tim = tim_ref[...]
    dot = lambda a, b: jnp.dot(a, b, preferred_element_type=jnp.float32)

    def stage1(x):
        br = dot(fre, x)
        bi = dot(fim, x)
        return br * tre - bi * tim, br * tim + bi * tre

    qcr, qci = stage1(q_ref[0, 0])
    qcr_ref[0, 0] = qcr
    qci_ref[0, 0] = qci
    kcr, kci = stage1(k_ref[0, 0])
    kcr_ref[0, 0] = kcr
    kci_ref[0, 0] = kci


def _f2_body(qcr_ref, qci_ref, kcr_ref, kci_ref, fre_ref, fim_ref,
             er_ref, ei_ref):
    fre = fre_ref[...]
    fim = fim_ref[...]
    dot = lambda a, b: jnp.dot(a, b, preferred_element_type=jnp.float32)

    def stage2(cr, ci):
        return dot(cr, fre) - dot(ci, fim), dot(cr, fim) + dot(ci, fre)

    qr, qi = stage2(qcr_ref[0, 0], qci_ref[0, 0])
    kr, ki = stage2(kcr_ref[0, 0], kci_ref[0, 0])
    sr = qr * kr + qi * ki
    si = qi * kr - qr * ki
    er_ref[0, 0] = dot(sr, fre) + dot(si, fim)
    ei_ref[0, 0] = dot(si, fre) - dot(sr, fim)


def _f3_body(er_ref, ei_ref, fre_ref, fim_ref, tre_ref, tim_ref, jones_ref,
             corr_ref, r_ref):
    jb = pl.program_id(1)
    fre = fre_ref[...]
    fim = fim_ref[...]
    tre = tre_ref[...]
    tim = tim_ref[...]
    dot = lambda a, b: jnp.dot(a, b, preferred_element_type=jnp.float32)
    er = er_ref[0, 0]
    ei = ei_ref[0, 0]
    cr = er * tre + ei * tim
    ci = ei * tre - er * tim
    ar = (dot(fre, cr) + dot(fim, ci)) * (1.0 / 4096.0)
    corr_ref[0, 0] = ar
    rblk = dot(ar, jones_ref[...])

    @pl.when(jb == 0)
    def _():
        r_ref[0] = rblk

    @pl.when(jb != 0)
    def _():
        r_ref[0] = r_ref[0] + rblk


def _topk_body(r_ref, tau_ref, wgt_ref):
    rm = r_ref[0] + r_ref[1] + r_ref[2] + r_ref[3]
    row = jax.lax.broadcasted_iota(jnp.int32, (64, 64), 0)
    col = jax.lax.broadcasted_iota(jnp.int32, (64, 64), 1)
    lin = row * 64 + col
    big = jnp.int32(1 << 30)
    neg = jnp.float32(-3.0e38)
    tr = [[None] * _KT for _ in range(_B)]
    for i in range(_KT):
        m = jnp.max(rm)
        idx = jnp.min(jnp.where(rm == m, lin, big))
        tau_ref[i] = idx
        sel = lin == idx
        for b in range(_B):
            tr[b][i] = jnp.sum(jnp.where(sel, r_ref[b], 0.0)) * (1.0 / 768.0)
        rm = jnp.where(sel, neg, rm)
    for b in range(_B):
        mx = tr[b][0]
        for i in range(1, _KT):
            mx = jnp.maximum(mx, tr[b][i])
        es = [jnp.exp(tr[b][i] - mx) for i in range(_KT)]
        tot = es[0]
        for i in range(1, _KT):
            tot = tot + es[i]
        inv = 1.0 / tot
        for i in range(_KT):
            wgt_ref[b, i] = es[i] * inv


def _agg_body(tau_ref, wgt_ref, vo_ref, out_ref, *scratch):
    bufs = scratch[:_KT]
    sems = scratch[_KT]
    b = pl.program_id(0)
    l0 = pl.program_id(1) * _LB
    copies = []
    for i in range(_KT):
        start = pl.multiple_of((b * 2 * _L + l0 + tau_ref[i]) * _D, _D)
        cp = pltpu.make_async_copy(
            vo_ref.at[pl.ds(start, _LB * _D)], bufs[i], sems.at[i])
        cp.start()
        copies.append(cp)
    acc = None
    for i in range(_KT):
        copies[i].wait()
        term = wgt_ref[b, i] * bufs[i][...]
        acc = term if acc is None else acc + term
    out_ref[...] = acc


def kernel(Q, K, V, Wq, bq, Wk, bk, Wv, bv, Wo, bo):
    f32 = jnp.float32
    fre = jnp.asarray(_F_RE)
    fim = jnp.asarray(_F_IM)
    tre = jnp.asarray(_TB_RE)
    tim = jnp.asarray(_TB_IM)
    jones = jnp.asarray(_JONES)

    q2 = Q.reshape(_B * _L, _D)
    k2 = K.reshape(_B * _L, _D)
    v2 = V.reshape(_B * _L, _D)
    wqt = Wq.T
    wkt = Wk.T
    wvt = Wv.T
    wot = Wo.T
    bias = jnp.stack([bq, bk, bv, bo])

    nrb = (_B * _L) // _RB
    row_spec = pl.BlockSpec((_RB, _D), lambda i: (i, 0))
    w_spec = pl.BlockSpec((_D, _D), lambda i: (0, 0))
    b_spec = pl.BlockSpec((4, _D), lambda i: (0, 0))
    qf, kf, vof = pl.pallas_call(
        _proj_body,
        grid=(nrb,),
        in_specs=[row_spec, row_spec, row_spec, w_spec, w_spec, w_spec,
                  w_spec, b_spec],
        out_specs=[row_spec, row_spec, row_spec],
        out_shape=[jax.ShapeDtypeStruct((_B * _L, _D), f32)] * 3,
    )(q2, k2, v2, wqt, wkt, wvt, wot, bias)

    def to_fft_layout(x):
        x4 = x.reshape(_B, _N1, _N2, _NJ, _JB)
        return x4.transpose(0, 3, 1, 4, 2).reshape(_B, _NJ, 64, _JB * 64)

    qA = to_fft_layout(qf)
    kA = to_fft_layout(kf)

    blk_spec = pl.BlockSpec((1, 1, 64, _JB * 64), lambda b, j: (b, j, 0, 0))
    c_spec = pl.BlockSpec((64, 64), lambda b, j: (0, 0))
    t_spec = pl.BlockSpec((64, _JB * 64), lambda b, j: (0, 0))
    fshape = jax.ShapeDtypeStruct((_B, _NJ, 64, _JB * 64), f32)

    qcr, qci, kcr, kci = pl.pallas_call(
        _f1_body,
        grid=(_B, _NJ),
        in_specs=[blk_spec, blk_spec, c_spec, c_spec, t_spec, t_spec],
        out_specs=[blk_spec] * 4,
        out_shape=[fshape] * 4,
    )(qA, kA, fre, fim, tre, tim)

    rs = lambda x: x.reshape(_B, _NJ, 64 * _JB, 64)
    blk2_spec = pl.BlockSpec((1, 1, 64 * _JB, 64), lambda b, j: (b, j, 0, 0))
    f2shape = jax.ShapeDtypeStruct((_B, _NJ, 64 * _JB, 64), f32)
    er, ei = pl.pallas_call(
        _f2_body,
        grid=(_B, _NJ),
        in_specs=[blk2_spec] * 4 + [c_spec, c_spec],
        out_specs=[blk2_spec] * 2,
        out_shape=[f2shape] * 2,
    )(rs(qcr), rs(qci), rs(kcr), rs(kci), fre, fim)

    jones_spec = pl.BlockSpec((64 * _JB, 64), lambda b, j: (0, 0))
    r_spec = pl.BlockSpec((1, 64, 64), lambda b, j: (b, 0, 0))
    corr5, r_out = pl.pallas_call(
        _f3_body,
        grid=(_B, _NJ),
        in_specs=[blk_spec, blk_spec, c_spec, c_spec, t_spec, t_spec,
                  jones_spec],
        out_specs=[blk_spec, r_spec],
        out_shape=[fshape, jax.ShapeDtypeStruct((_B, 64, 64), f32)],
        compiler_params=pltpu.CompilerParams(
            dimension_semantics=("arbitrary", "arbitrary")),
    )(er.reshape(_B, _NJ, 64, _JB * 64), ei.reshape(_B, _NJ, 64, _JB * 64),
      fre, fim, tre, tim, jones)

    corr_out = (corr5.reshape(_B, _NJ, _N1, _JB, _N2)
                .transpose(0, 2, 4, 1, 3)
                .reshape(_B, _L, _NH, _DK))

    tau, wgt = pl.pallas_call(
        _topk_body,
        grid=(1,),
        in_specs=[pl.BlockSpec((_B, 64, 64), lambda i: (0, 0, 0))],
        out_specs=[pl.BlockSpec(memory_space=pltpu.SMEM),
                   pl.BlockSpec(memory_space=pltpu.SMEM)],
        out_shape=[jax.ShapeDtypeStruct((_KT,), jnp.int32),
                   jax.ShapeDtypeStruct((_B, _KT), f32)],
    )(r_out)

    vo3 = vof.reshape(_B, _L, _D)
    vo_flat = jnp.concatenate([vo3, vo3], axis=1).reshape(_B * 2 * _L * _D)

    nlb = _L // _LB
    out_flat = pl.pallas_call(
        _agg_body,
        grid=(_B, nlb),
        in_specs=[pl.BlockSpec(memory_space=pltpu.SMEM),
                  pl.BlockSpec(memory_space=pltpu.SMEM),
                  pl.BlockSpec(memory_space=pl.ANY)],
        out_specs=pl.BlockSpec((_LB * _D,), lambda b, i: (b * nlb + i,)),
        out_shape=jax.ShapeDtypeStruct((_B * _L * _D,), f32),
        scratch_shapes=[pltpu.VMEM((_LB * _D,), f32)] * _KT
                       + [pltpu.SemaphoreType.DMA((_KT,))],
    )(tau, wgt, vo_flat)

    return out_flat.reshape(_B, _L, _D), corr_out


# speedup vs baseline: 1.9542x; 1.0435x over previous
import numpy as np
import jax
import jax.numpy as jnp
from jax.experimental import pallas as pl
from jax.experimental.pallas import tpu as pltpu

_B, _L, _D = 4, 4096, 768
_NH, _DK = 12, 64
_N1 = _N2 = 64
_NJ = 12
_JB = 64
_KT = 8
_RB = 512
_LB = 512

_ar = np.arange(64)
_F64 = np.exp(-2j * np.pi * np.outer(_ar, _ar) / 64.0)
_TW = np.exp(-2j * np.pi * np.outer(_ar, _ar) / 4096.0)
_TWB = np.tile(_TW, (1, _JB))
_F_RE = np.ascontiguousarray(_F64.real.astype(np.float32))
_F_IM = np.ascontiguousarray(_F64.imag.astype(np.float32))
_TB_RE = np.ascontiguousarray(_TWB.real.astype(np.float32))
_TB_IM = np.ascontiguousarray(_TWB.imag.astype(np.float32))
_JONES = np.tile(np.eye(64, dtype=np.float32), (_JB, 1))


def _proj_body(q_ref, k_ref, v_ref, wq_ref, wk_ref, wv_ref, wo_ref, b_ref,
               qo_ref, ko_ref, vo_ref):
    bq = b_ref[0, :]
    bk = b_ref[1, :]
    bv = b_ref[2, :]
    bo = b_ref[3, :]
    dot = lambda a, b: jnp.dot(a, b, preferred_element_type=jnp.float32)
    qo_ref[0] = dot(q_ref[0], wq_ref[...]) + bq[None, :]
    ko_ref[0] = dot(k_ref[0], wk_ref[...]) + bk[None, :]
    v = dot(v_ref[0], wv_ref[...]) + bv[None, :]
    vo = dot(v, wo_ref[...]) + bo[None, :]
    vo_ref[0, 0] = vo
    vo_ref[0, 1] = vo


def _f1_body(q_ref, k_ref, fre_ref, fim_ref, tre_ref, tim_ref,
             qcr_ref, qci_ref, kcr_ref, kci_ref):
    fre = fre_ref[...]
    fim = fim_ref[...]
    tre = tre_ref[...]
    tim = tim_ref[...]
    dot = lambda a, b: jnp.dot(a, b, preferred_element_type=jnp.float32)

    def stage1(x):
        br = dot(fre, x)
        bi = dot(fim, x)
        return br * tre - bi * tim, br * tim + bi * tre

    qcr, qci = stage1(q_ref[0, 0])
    qcr_ref[0, 0] = qcr
    qci_ref[0, 0] = qci
    kcr, kci = stage1(k_ref[0, 0])
    kcr_ref[0, 0] = kcr
    kci_ref[0, 0] = kci


def _f2_body(qcr_ref, qci_ref, kcr_ref, kci_ref, fre_ref, fim_ref,
             er_ref, ei_ref):
    fre = fre_ref[...]
    fim = fim_ref[...]
    dot = lambda a, b: jnp.dot(a, b, preferred_element_type=jnp.float32)

    def stage2(cr, ci):
        return dot(cr, fre) - dot(ci, fim), dot(cr, fim) + dot(ci, fre)

    qr, qi = stage2(qcr_ref[0, 0], qci_ref[0, 0])
    kr, ki = stage2(kcr_ref[0, 0], kci_ref[0, 0])
    sr = qr * kr + qi * ki
    si = qi * kr - qr * ki
    er_ref[0, 0] = dot(sr, fre) + dot(si, fim)
    ei_ref[0, 0] = dot(si, fre) - dot(sr, fim)


def _f3_body(er_ref, ei_ref, fre_ref, fim_ref, tre_ref, tim_ref, jones_ref,
             corr_ref, r_ref):
    jb = pl.program_id(1)
    fre = fre_ref[...]
    fim = fim_ref[...]
    tre = tre_ref[...]
    tim = tim_ref[...]
    dot = lambda a, b: jnp.dot(a, b, preferred_element_type=jnp.float32)
    er = er_ref[0, 0]
    ei = ei_ref[0, 0]
    cr = er * tre + ei * tim
    ci = ei * tre - er * tim
    ar = (dot(fre, cr) + dot(fim, ci)) * (1.0 / 4096.0)
    corr_ref[0, 0] = ar
    rblk = dot(ar, jones_ref[...])

    @pl.when(jb == 0)
    def _():
        r_ref[0] = rblk

    @pl.when(jb != 0)
    def _():
        r_ref[0] = r_ref[0] + rblk


def _topk_body(r_ref, tau_ref, wgt_ref):
    rm = r_ref[0] + r_ref[1] + r_ref[2] + r_ref[3]
    row = jax.lax.broadcasted_iota(jnp.int32, (64, 64), 0)
    col = jax.lax.broadcasted_iota(jnp.int32, (64, 64), 1)
    lin = row * 64 + col
    big = jnp.int32(1 << 30)
    neg = jnp.float32(-3.0e38)
    tr = [[None] * _KT for _ in range(_B)]
    for i in range(_KT):
        m = jnp.max(rm)
        idx = jnp.min(jnp.where(rm == m, lin, big))
        tau_ref[i] = idx
        sel = lin == idx
        for b in range(_B):
            tr[b][i] = jnp.sum(jnp.where(sel, r_ref[b], 0.0)) * (1.0 / 768.0)
        rm = jnp.where(sel, neg, rm)
    for b in range(_B):
        mx = tr[b][0]
        for i in range(1, _KT):
            mx = jnp.maximum(mx, tr[b][i])
        es = [jnp.exp(tr[b][i] - mx) for i in range(_KT)]
        tot = es[0]
        for i in range(1, _KT):
            tot = tot + es[i]
        inv = 1.0 / tot
        for i in range(_KT):
            wgt_ref[b, i] = es[i] * inv


def _agg_body(tau_ref, wgt_ref, vo_ref, out_ref, *scratch):
    bufs = scratch[:_KT]
    sems = scratch[_KT]
    b = pl.program_id(0)
    l0 = pl.program_id(1) * _LB
    copies = []
    for i in range(_KT):
        start = pl.multiple_of((b * 2 * _L + l0 + tau_ref[i]) * _D, _D)
        cp = pltpu.make_async_copy(
            vo_ref.at[pl.ds(start, _LB * _D)], bufs[i], sems.at[i])
        cp.start()
        copies.append(cp)
    acc = None
    for i in range(_KT):
        copies[i].wait()
        term = wgt_ref[b, i] * bufs[i][...]
        acc = term if acc is None else acc + term
    out_ref[...] = acc


def kernel(Q, K, V, Wq, bq, Wk, bk, Wv, bv, Wo, bo):
    f32 = jnp.float32
    fre = jnp.asarray(_F_RE)
    fim = jnp.asarray(_F_IM)
    tre = jnp.asarray(_TB_RE)
    tim = jnp.asarray(_TB_IM)
    jones = jnp.asarray(_JONES)

    wqt = Wq.T
    wkt = Wk.T
    wvt = Wv.T
    wot = Wo.T
    bias = jnp.stack([bq, bk, bv, bo])

    nrb = _L // _RB
    row_spec = pl.BlockSpec((1, _RB, _D), lambda b, i: (b, i, 0))
    w_spec = pl.BlockSpec((_D, _D), lambda b, i: (0, 0))
    b_spec = pl.BlockSpec((4, _D), lambda b, i: (0, 0))
    vo_spec = pl.BlockSpec((1, 2, _RB, _D), lambda b, i: (b, 0, i, 0))
    qf, kf, vop = pl.pallas_call(
        _proj_body,
        grid=(_B, nrb),
        in_specs=[row_spec, row_spec, row_spec, w_spec, w_spec, w_spec,
                  w_spec, b_spec],
        out_specs=[row_spec, row_spec, vo_spec],
        out_shape=[jax.ShapeDtypeStruct((_B, _L, _D), f32)] * 2
                  + [jax.ShapeDtypeStruct((_B, 2, _L, _D), f32)],
    )(Q, K, V, wqt, wkt, wvt, wot, bias)

    def to_fft_layout(x):
        x4 = x.reshape(_B, _N1, _N2, _NJ, _JB)
        return x4.transpose(0, 3, 1, 4, 2).reshape(_B, _NJ, 64, _JB * 64)

    qA = to_fft_layout(qf)
    kA = to_fft_layout(kf)

    blk_spec = pl.BlockSpec((1, 1, 64, _JB * 64), lambda b, j: (b, j, 0, 0))
    c_spec = pl.BlockSpec((64, 64), lambda b, j: (0, 0))
    t_spec = pl.BlockSpec((64, _JB * 64), lambda b, j: (0, 0))
    fshape = jax.ShapeDtypeStruct((_B, _NJ, 64, _JB * 64), f32)

    qcr, qci, kcr, kci = pl.pallas_call(
        _f1_body,
        grid=(_B, _NJ),
        in_specs=[blk_spec, blk_spec, c_spec, c_spec, t_spec, t_spec],
        out_specs=[blk_spec] * 4,
        out_shape=[fshape] * 4,
    )(qA, kA, fre, fim, tre, tim)

    rs = lambda x: x.reshape(_B, _NJ, 64 * _JB, 64)
    blk2_spec = pl.BlockSpec((1, 1, 64 * _JB, 64), lambda b, j: (b, j, 0, 0))
    f2shape = jax.ShapeDtypeStruct((_B, _NJ, 64 * _JB, 64), f32)
    er, ei = pl.pallas_call(
        _f2_body,
        grid=(_B, _NJ),
        in_specs=[blk2_spec] * 4 + [c_spec, c_spec],
        out_specs=[blk2_spec] * 2,
        out_shape=[f2shape] * 2,
    )(rs(qcr), rs(qci), rs(kcr), rs(kci), fre, fim)

    jones_spec = pl.BlockSpec((64 * _JB, 64), lambda b, j: (0, 0))
    r_spec = pl.BlockSpec((1, 64, 64), lambda b, j: (b, 0, 0))
    corr5, r_out = pl.pallas_call(
        _f3_body,
        grid=(_B, _NJ),
        in_specs=[blk_spec, blk_spec, c_spec, c_spec, t_spec, t_spec,
                  jones_spec],
        out_specs=[blk_spec, r_spec],
        out_shape=[fshape, jax.ShapeDtypeStruct((_B, 64, 64), f32)],
        compiler_params=pltpu.CompilerParams(
            dimension_semantics=("arbitrary", "arbitrary")),
    )(er.reshape(_B, _NJ, 64, _JB * 64), ei.reshape(_B, _NJ, 64, _JB * 64),
      fre, fim, tre, tim, jones)

    corr_out = (corr5.reshape(_B, _NJ, _N1, _JB, _N2)
                .transpose(0, 2, 4, 1, 3)
                .reshape(_B, _L, _NH, _DK))

    tau, wgt = pl.pallas_call(
        _topk_body,
        grid=(1,),
        in_specs=[pl.BlockSpec((_B, 64, 64), lambda i: (0, 0, 0))],
        out_specs=[pl.BlockSpec(memory_space=pltpu.SMEM),
                   pl.BlockSpec(memory_space=pltpu.SMEM)],
        out_shape=[jax.ShapeDtypeStruct((_KT,), jnp.int32),
                   jax.ShapeDtypeStruct((_B, _KT), f32)],
    )(r_out)

    vo_flat = vop.reshape(_B * 2 * _L * _D)

    nlb = _L // _LB
    out_flat = pl.pallas_call(
        _agg_body,
        grid=(_B, nlb),
        in_specs=[pl.BlockSpec(memory_space=pltpu.SMEM),
                  pl.BlockSpec(memory_space=pltpu.SMEM),
                  pl.BlockSpec(memory_space=pl.ANY)],
        out_specs=pl.BlockSpec((_LB * _D,), lambda b, i: (b * nlb + i,)),
        out_shape=jax.ShapeDtypeStruct((_B * _L * _D,), f32),
        scratch_shapes=[pltpu.VMEM((_LB * _D,), f32)] * _KT
                       + [pltpu.SemaphoreType.DMA((_KT,))],
    )(tau, wgt, vo_flat)

    return out_flat.reshape(_B, _L, _D), corr_out


# speedup vs baseline: 2.8919x; 1.4798x over previous
import numpy as np
import jax
import jax.numpy as jnp
from jax.experimental import pallas as pl
from jax.experimental.pallas import tpu as pltpu

_B, _L, _D = 4, 4096, 768
_NH, _DK = 12, 64
_N1 = _N2 = 64
_NJ = 12
_JB = 64
_KT = 8
_RB = 512
_LB = 512

_NG = 8
_GW = (_N2 // _NG) * _D

_ar = np.arange(64)
_F64 = np.exp(-2j * np.pi * np.outer(_ar, _ar) / 64.0)
_TW = np.exp(-2j * np.pi * np.outer(_ar, _ar) / 4096.0)
_F_RE = np.ascontiguousarray(_F64.real.astype(np.float32))
_F_IM = np.ascontiguousarray(_F64.imag.astype(np.float32))
_TB3_RE = np.ascontiguousarray(np.repeat(_TW.real, _D, axis=1).astype(np.float32))
_TB3_IM = np.ascontiguousarray(np.repeat(_TW.imag, _D, axis=1).astype(np.float32))
_JONES8 = np.zeros((_GW, 64), dtype=np.float32)
_JONES8[np.arange(_GW), np.arange(_GW) // _D] = 1.0


def _proj_body(q_ref, k_ref, v_ref, wq_ref, wk_ref, wv_ref, wo_ref, b_ref,
               qo_ref, ko_ref, vo_ref):
    bq = b_ref[0, :]
    bk = b_ref[1, :]
    bv = b_ref[2, :]
    bo = b_ref[3, :]
    dot = lambda a, b: jnp.dot(a, b, preferred_element_type=jnp.float32)
    qo_ref[0] = dot(q_ref[0], wq_ref[...]) + bq[None, :]
    ko_ref[0] = dot(k_ref[0], wk_ref[...]) + bk[None, :]
    v = dot(v_ref[0], wv_ref[...]) + bv[None, :]
    vo = dot(v, wo_ref[...]) + bo[None, :]
    vo_ref[0, 0] = vo
    vo_ref[0, 1] = vo


def _f1_body(q_ref, k_ref, fre_ref, fim_ref, tre_ref, tim_ref,
             qcr_ref, qci_ref, kcr_ref, kci_ref):
    fre = fre_ref[...]
    fim = fim_ref[...]
    tre = tre_ref[...]
    tim = tim_ref[...]
    dot = lambda a, b: jnp.dot(a, b, preferred_element_type=jnp.float32)

    def stage1(x):
        br = dot(fre, x)
        bi = dot(fim, x)
        return br * tre - bi * tim, br * tim + bi * tre

    qcr, qci = stage1(q_ref[0])
    qcr_ref[0] = qcr
    qci_ref[0] = qci
    kcr, kci = stage1(k_ref[0])
    kcr_ref[0] = kcr
    kci_ref[0] = kci


def _f2_body(qcr_ref, qci_ref, kcr_ref, kci_ref, fre_ref, fim_ref,
             er_ref, ei_ref):
    fre = fre_ref[...]
    fim = fim_ref[...]
    dot = lambda a, b: jnp.dot(a, b, preferred_element_type=jnp.float32)

    def stage2(cr, ci):
        return dot(fre, cr) - dot(fim, ci), dot(fim, cr) + dot(fre, ci)

    qr, qi = stage2(qcr_ref[0, 0], qci_ref[0, 0])
    kr, ki = stage2(kcr_ref[0, 0], kci_ref[0, 0])
    sr = qr * kr + qi * ki
    si = qi * kr - qr * ki
    er_ref[0, 0] = dot(fre, sr) + dot(fim, si)
    ei_ref[0, 0] = dot(fre, si) - dot(fim, sr)


def _f3_body(er_ref, ei_ref, fre_ref, fim_ref, tre_ref, tim_ref, jones_ref,
             corr_ref, rp_ref):
    fre = fre_ref[...]
    fim = fim_ref[...]
    tre = tre_ref[...]
    tim = tim_ref[...]
    dot = lambda a, b: jnp.dot(a, b, preferred_element_type=jnp.float32)
    er = er_ref[0]
    ei = ei_ref[0]
    cr = er * tre + ei * tim
    ci = ei * tre - er * tim
    ar = (dot(fre, cr) + dot(fim, ci)) * (1.0 / 4096.0)
    corr_ref[0] = ar
    rp_ref[0] = dot(ar, jones_ref[...])


def _topk_body(r_ref, tau_ref, wgt_ref):
    rm = r_ref[0] + r_ref[1] + r_ref[2] + r_ref[3]
    row = jax.lax.broadcasted_iota(jnp.int32, (64, 64), 0)
    col = jax.lax.broadcasted_iota(jnp.int32, (64, 64), 1)
    lin = row * 64 + col
    big = jnp.int32(1 << 30)
    neg = jnp.float32(-3.0e38)
    tr = [[None] * _KT for _ in range(_B)]
    for i in range(_KT):
        m = jnp.max(rm)
        idx = jnp.min(jnp.where(rm == m, lin, big))
        tau_ref[i] = idx
        sel = lin == idx
        for b in range(_B):
            tr[b][i] = jnp.sum(jnp.where(sel, r_ref[b], 0.0)) * (1.0 / 768.0)
        rm = jnp.where(sel, neg, rm)
    for b in range(_B):
        mx = tr[b][0]
        for i in range(1, _KT):
            mx = jnp.maximum(mx, tr[b][i])
        es = [jnp.exp(tr[b][i] - mx) for i in range(_KT)]
        tot = es[0]
        for i in range(1, _KT):
            tot = tot + es[i]
        inv = 1.0 / tot
        for i in range(_KT):
            wgt_ref[b, i] = es[i] * inv


def _agg_body(tau_ref, wgt_ref, vo_ref, out_ref, *scratch):
    bufs = scratch[:_KT]
    sems = scratch[_KT]
    b = pl.program_id(0)
    l0 = pl.program_id(1) * _LB
    copies = []
    for i in range(_KT):
        start = pl.multiple_of((b * 2 * _L + l0 + tau_ref[i]) * _D, _D)
        cp = pltpu.make_async_copy(
            vo_ref.at[pl.ds(start, _LB * _D)], bufs[i], sems.at[i])
        cp.start()
        copies.append(cp)
    acc = None
    for i in range(_KT):
        copies[i].wait()
        term = wgt_ref[b, i] * bufs[i][...]
        acc = term if acc is None else acc + term
    out_ref[...] = acc


def kernel(Q, K, V, Wq, bq, Wk, bk, Wv, bv, Wo, bo):
    f32 = jnp.float32
    fre = jnp.asarray(_F_RE)
    fim = jnp.asarray(_F_IM)
    tre = jnp.asarray(_TB3_RE)
    tim = jnp.asarray(_TB3_IM)
    jones = jnp.asarray(_JONES8)

    wqt = Wq.T
    wkt = Wk.T
    wvt = Wv.T
    wot = Wo.T
    bias = jnp.stack([bq, bk, bv, bo])

    nrb = _L // _RB
    row_spec = pl.BlockSpec((1, _RB, _D), lambda b, i: (b, i, 0))
    w_spec = pl.BlockSpec((_D, _D), lambda b, i: (0, 0))
    b_spec = pl.BlockSpec((4, _D), lambda b, i: (0, 0))
    vo_spec = pl.BlockSpec((1, 2, _RB, _D), lambda b, i: (b, 0, i, 0))
    qf, kf, vop = pl.pallas_call(
        _proj_body,
        grid=(_B, nrb),
        in_specs=[row_spec, row_spec, row_spec, w_spec, w_spec, w_spec,
                  w_spec, b_spec],
        out_specs=[row_spec, row_spec, vo_spec],
        out_shape=[jax.ShapeDtypeStruct((_B, _L, _D), f32)] * 2
                  + [jax.ShapeDtypeStruct((_B, 2, _L, _D), f32)],
    )(Q, K, V, wqt, wkt, wvt, wot, bias)

    qn = qf.reshape(_B, _N1, _N2 * _D)
    kn = kf.reshape(_B, _N1, _N2 * _D)

    g_spec = pl.BlockSpec((1, 64, _GW), lambda g, b: (b, 0, g))
    c_spec = pl.BlockSpec((64, 64), lambda g, b: (0, 0))
    t_spec = pl.BlockSpec((64, _GW), lambda g, b: (0, g))
    fshape = jax.ShapeDtypeStruct((_B, _N1, _N2 * _D), f32)

    qcr, qci, kcr, kci = pl.pallas_call(
        _f1_body,
        grid=(_NG, _B),
        in_specs=[g_spec, g_spec, c_spec, c_spec, t_spec, t_spec],
        out_specs=[g_spec] * 4,
        out_shape=[fshape] * 4,
    )(qn, kn, fre, fim, tre, tim)

    rs = lambda x: x.reshape(_B, _N1, _N2, _D)
    k_spec = pl.BlockSpec((1, 1, 64, _D), lambda k1, b: (b, k1, 0, 0))
    c2_spec = pl.BlockSpec((64, 64), lambda k1, b: (0, 0))
    f2shape = jax.ShapeDtypeStruct((_B, _N1, _N2, _D), f32)
    er, ei = pl.pallas_call(
        _f2_body,
        grid=(_N1, _B),
        in_specs=[k_spec] * 4 + [c2_spec, c2_spec],
        out_specs=[k_spec] * 2,
        out_shape=[f2shape] * 2,
    )(rs(qcr), rs(qci), rs(kcr), rs(kci), fre, fim)

    jones_spec = pl.BlockSpec((_GW, 64), lambda g, b: (0, 0))
    rp_spec = pl.BlockSpec((1, 64, 64), lambda g, b: (b * _NG + g, 0, 0))
    corr_n, r_part = pl.pallas_call(
        _f3_body,
        grid=(_NG, _B),
        in_specs=[g_spec, g_spec, c_spec, c_spec, t_spec, t_spec,
                  jones_spec],
        out_specs=[g_spec, rp_spec],
        out_shape=[fshape, jax.ShapeDtypeStruct((_B * _NG, 64, 64), f32)],
    )(er.reshape(_B, _N1, _N2 * _D), ei.reshape(_B, _N1, _N2 * _D),
      fre, fim, tre, tim, jones)
    r_part = r_part.reshape(_B, _NG, 64, 64)

    corr_out = corr_n.reshape(_B, _L, _NH, _DK)
    r_out = (r_part[:, :, :, :_NG].transpose(0, 2, 1, 3)
             .reshape(_B, 64, 64))

    tau, wgt = pl.pallas_call(
        _topk_body,
        grid=(1,),
        in_specs=[pl.BlockSpec((_B, 64, 64), lambda i: (0, 0, 0))],
        out_specs=[pl.BlockSpec(memory_space=pltpu.SMEM),
                   pl.BlockSpec(memory_space=pltpu.SMEM)],
        out_shape=[jax.ShapeDtypeStruct((_KT,), jnp.int32),
                   jax.ShapeDtypeStruct((_B, _KT), f32)],
    )(r_out)

    vo_flat = vop.reshape(_B * 2 * _L * _D)

    nlb = _L // _LB
    out_flat = pl.pallas_call(
        _agg_body,
        grid=(_B, nlb),
        in_specs=[pl.BlockSpec(memory_space=pltpu.SMEM),
                  pl.BlockSpec(memory_space=pltpu.SMEM),
                  pl.BlockSpec(memory_space=pl.ANY)],
        out_specs=pl.BlockSpec((_LB * _D,), lambda b, i: (b * nlb + i,)),
        out_shape=jax.ShapeDtypeStruct((_B * _L * _D,), f32),
        scratch_shapes=[pltpu.VMEM((_LB * _D,), f32)] * _KT
                       + [pltpu.SemaphoreType.DMA((_KT,))],
    )(tau, wgt, vo_flat)

    return out_flat.reshape(_B, _L, _D), corr_out


# speedup vs baseline: 3.1389x; 1.0854x over previous
import numpy as np
import jax
import jax.numpy as jnp
from jax.experimental import pallas as pl
from jax.experimental.pallas import tpu as pltpu

_B, _L, _D = 4, 4096, 768
_NH, _DK = 12, 64
_N1 = _N2 = 64
_NJ = 12
_JB = 64
_KT = 8
_RB = 512
_LB = 512

_NG = 8
_K1C = 4
_GW = (_N2 // _NG) * _D

_ar = np.arange(64)
_F64 = np.exp(-2j * np.pi * np.outer(_ar, _ar) / 64.0)
_TW = np.exp(-2j * np.pi * np.outer(_ar, _ar) / 4096.0)
_F_RE = np.ascontiguousarray(_F64.real.astype(np.float32))
_F_IM = np.ascontiguousarray(_F64.imag.astype(np.float32))
_TB3_RE = np.ascontiguousarray(np.repeat(_TW.real, _D, axis=1).astype(np.float32))
_TB3_IM = np.ascontiguousarray(np.repeat(_TW.imag, _D, axis=1).astype(np.float32))
_JONES8 = np.zeros((_GW, 64), dtype=np.float32)
_JONES8[np.arange(_GW), np.arange(_GW) // _D] = 1.0


def _proj_body(q_ref, k_ref, v_ref, wq_ref, wk_ref, wv_ref, wo_ref, b_ref,
               qo_ref, ko_ref, vo_ref):
    bq = b_ref[0, :]
    bk = b_ref[1, :]
    bv = b_ref[2, :]
    bo = b_ref[3, :]
    dot = lambda a, b: jnp.dot(a, b, preferred_element_type=jnp.float32)
    qo_ref[0] = dot(q_ref[0], wq_ref[...]) + bq[None, :]
    ko_ref[0] = dot(k_ref[0], wk_ref[...]) + bk[None, :]
    v = dot(v_ref[0], wv_ref[...]) + bv[None, :]
    vo = dot(v, wo_ref[...]) + bo[None, :]
    vo_ref[0, 0] = vo
    vo_ref[0, 1] = vo


def _f1_body(q_ref, k_ref, fre_ref, fim_ref, tre_ref, tim_ref,
             qcr_ref, qci_ref, kcr_ref, kci_ref):
    fre = fre_ref[...]
    fim = fim_ref[...]
    tre = tre_ref[...]
    tim = tim_ref[...]
    dot = lambda a, b: jnp.dot(a, b, preferred_element_type=jnp.float32)

    def stage1(x):
        br = dot(fre, x)
        bi = dot(fim, x)
        return br * tre - bi * tim, br * tim + bi * tre

    qcr, qci = stage1(q_ref[0])
    qcr_ref[0] = qcr
    qci_ref[0] = qci
    kcr, kci = stage1(k_ref[0])
    kcr_ref[0] = kcr
    kci_ref[0] = kci


def _f2_body(qcr_ref, qci_ref, kcr_ref, kci_ref, fre_ref, fim_ref,
             er_ref, ei_ref):
    fre = fre_ref[...]
    fim = fim_ref[...]
    dot = lambda a, b: jnp.dot(a, b, preferred_element_type=jnp.float32)

    def stage2(cr, ci):
        return dot(fre, cr) - dot(fim, ci), dot(fim, cr) + dot(fre, ci)

    for t in range(_K1C):
        qr, qi = stage2(qcr_ref[0, t], qci_ref[0, t])
        kr, ki = stage2(kcr_ref[0, t], kci_ref[0, t])
        sr = qr * kr + qi * ki
        si = qi * kr - qr * ki
        er_ref[0, t] = dot(fre, sr) + dot(fim, si)
        ei_ref[0, t] = dot(fre, si) - dot(fim, sr)


def _f3_body(er_ref, ei_ref, fre_ref, fim_ref, tre_ref, tim_ref, jones_ref,
             corr_ref, rp_ref):
    fre = fre_ref[...]
    fim = fim_ref[...]
    tre = tre_ref[...]
    tim = tim_ref[...]
    dot = lambda a, b: jnp.dot(a, b, preferred_element_type=jnp.float32)
    er = er_ref[0]
    ei = ei_ref[0]
    cr = er * tre + ei * tim
    ci = ei * tre - er * tim
    ar = (dot(fre, cr) + dot(fim, ci)) * (1.0 / 4096.0)
    corr_ref[0] = ar
    rp_ref[0] = dot(ar, jones_ref[...])


def _topk_body(r_ref, tau_ref, wgt_ref):
    rm = r_ref[0] + r_ref[1] + r_ref[2] + r_ref[3]
    row = jax.lax.broadcasted_iota(jnp.int32, (64, 64), 0)
    col = jax.lax.broadcasted_iota(jnp.int32, (64, 64), 1)
    lin = row * 64 + col
    big = jnp.int32(1 << 30)
    neg = jnp.float32(-3.0e38)
    tr = [[None] * _KT for _ in range(_B)]
    for i in range(_KT):
        m = jnp.max(rm)
        idx = jnp.min(jnp.where(rm == m, lin, big))
        tau_ref[i] = idx
        sel = lin == idx
        for b in range(_B):
            tr[b][i] = jnp.sum(jnp.where(sel, r_ref[b], 0.0)) * (1.0 / 768.0)
        rm = jnp.where(sel, neg, rm)
    for b in range(_B):
        mx = tr[b][0]
        for i in range(1, _KT):
            mx = jnp.maximum(mx, tr[b][i])
        es = [jnp.exp(tr[b][i] - mx) for i in range(_KT)]
        tot = es[0]
        for i in range(1, _KT):
            tot = tot + es[i]
        inv = 1.0 / tot
        for i in range(_KT):
            wgt_ref[b, i] = es[i] * inv


def _agg_body(tau_ref, wgt_ref, vo_ref, out_ref, *scratch):
    bufs = scratch[:_KT]
    sems = scratch[_KT]
    b = pl.program_id(0)
    l0 = pl.program_id(1) * _LB
    copies = []
    for i in range(_KT):
        start = pl.multiple_of((b * 2 * _L + l0 + tau_ref[i]) * _D, _D)
        cp = pltpu.make_async_copy(
            vo_ref.at[pl.ds(start, _LB * _D)], bufs[i], sems.at[i])
        cp.start()
        copies.append(cp)
    acc = None
    for i in range(_KT):
        copies[i].wait()
        term = wgt_ref[b, i] * bufs[i][...]
        acc = term if acc is None else acc + term
    out_ref[...] = acc


def kernel(Q, K, V, Wq, bq, Wk, bk, Wv, bv, Wo, bo):
    f32 = jnp.float32
    fre = jnp.asarray(_F_RE)
    fim = jnp.asarray(_F_IM)
    tre = jnp.asarray(_TB3_RE)
    tim = jnp.asarray(_TB3_IM)
    jones = jnp.asarray(_JONES8)

    wqt = Wq.T
    wkt = Wk.T
    wvt = Wv.T
    wot = Wo.T
    bias = jnp.stack([bq, bk, bv, bo])

    nrb = _L // _RB
    row_spec = pl.BlockSpec((1, _RB, _D), lambda b, i: (b, i, 0))
    w_spec = pl.BlockSpec((_D, _D), lambda b, i: (0, 0))
    b_spec = pl.BlockSpec((4, _D), lambda b, i: (0, 0))
    vo_spec = pl.BlockSpec((1, 2, _RB, _D), lambda b, i: (b, 0, i, 0))
    qf, kf, vop = pl.pallas_call(
        _proj_body,
        grid=(_B, nrb),
        in_specs=[row_spec, row_spec, row_spec, w_spec, w_spec, w_spec,
                  w_spec, b_spec],
        out_specs=[row_spec, row_spec, vo_spec],
        out_shape=[jax.ShapeDtypeStruct((_B, _L, _D), f32)] * 2
                  + [jax.ShapeDtypeStruct((_B, 2, _L, _D), f32)],
    )(Q, K, V, wqt, wkt, wvt, wot, bias)

    qn = qf.reshape(_B, _N1, _N2 * _D)
    kn = kf.reshape(_B, _N1, _N2 * _D)

    g_spec = pl.BlockSpec((1, 64, _GW), lambda g, b: (b, 0, g))
    c_spec = pl.BlockSpec((64, 64), lambda g, b: (0, 0))
    t_spec = pl.BlockSpec((64, _GW), lambda g, b: (0, g))
    fshape = jax.ShapeDtypeStruct((_B, _N1, _N2 * _D), f32)

    qcr, qci, kcr, kci = pl.pallas_call(
        _f1_body,
        grid=(_NG, _B),
        in_specs=[g_spec, g_spec, c_spec, c_spec, t_spec, t_spec],
        out_specs=[g_spec] * 4,
        out_shape=[fshape] * 4,
    )(qn, kn, fre, fim, tre, tim)

    rs = lambda x: x.reshape(_B, _N1, _N2, _D)
    k_spec = pl.BlockSpec((1, _K1C, 64, _D), lambda k1, b: (b, k1, 0, 0))
    c2_spec = pl.BlockSpec((64, 64), lambda k1, b: (0, 0))
    f2shape = jax.ShapeDtypeStruct((_B, _N1, _N2, _D), f32)
    er, ei = pl.pallas_call(
        _f2_body,
        grid=(_N1 // _K1C, _B),
        in_specs=[k_spec] * 4 + [c2_spec, c2_spec],
        out_specs=[k_spec] * 2,
        out_shape=[f2shape] * 2,
    )(rs(qcr), rs(qci), rs(kcr), rs(kci), fre, fim)

    jones_spec = pl.BlockSpec((_GW, 64), lambda g, b: (0, 0))
    rp_spec = pl.BlockSpec((1, 64, 64), lambda g, b: (b * _NG + g, 0, 0))
    corr_n, r_part = pl.pallas_call(
        _f3_body,
        grid=(_NG, _B),
        in_specs=[g_spec, g_spec, c_spec, c_spec, t_spec, t_spec,
                  jones_spec],
        out_specs=[g_spec, rp_spec],
        out_shape=[fshape, jax.ShapeDtypeStruct((_B * _NG, 64, 64), f32)],
    )(er.reshape(_B, _N1, _N2 * _D), ei.reshape(_B, _N1, _N2 * _D),
      fre, fim, tre, tim, jones)
    r_part = r_part.reshape(_B, _NG, 64, 64)

    corr_out = corr_n.reshape(_B, _L, _NH, _DK)
    r_out = (r_part[:, :, :, :_NG].transpose(0, 2, 1, 3)
             .reshape(_B, 64, 64))

    tau, wgt = pl.pallas_call(
        _topk_body,
        grid=(1,),
        in_specs=[pl.BlockSpec((_B, 64, 64), lambda i: (0, 0, 0))],
        out_specs=[pl.BlockSpec(memory_space=pltpu.SMEM),
                   pl.BlockSpec(memory_space=pltpu.SMEM)],
        out_shape=[jax.ShapeDtypeStruct((_KT,), jnp.int32),
                   jax.ShapeDtypeStruct((_B, _KT), f32)],
    )(r_out)

    vo_flat = vop.reshape(_B * 2 * _L * _D)

    nlb = _L // _LB
    out_flat = pl.pallas_call(
        _agg_body,
        grid=(_B, nlb),
        in_specs=[pl.BlockSpec(memory_space=pltpu.SMEM),
                  pl.BlockSpec(memory_space=pltpu.SMEM),
                  pl.BlockSpec(memory_space=pl.ANY)],
        out_specs=pl.BlockSpec((_LB * _D,), lambda b, i: (b * nlb + i,)),
        out_shape=jax.ShapeDtypeStruct((_B * _L * _D,), f32),
        scratch_shapes=[pltpu.VMEM((_LB * _D,), f32)] * _KT
                       + [pltpu.SemaphoreType.DMA((_KT,))],
    )(tau, wgt, vo_flat)

    return out_flat.reshape(_B, _L, _D), corr_out


# speedup vs baseline: 3.2473x; 1.0345x over previous
import numpy as np
import jax
import jax.numpy as jnp
from jax.experimental import pallas as pl
from jax.experimental.pallas import tpu as pltpu

_B, _L, _D = 4, 4096, 768
_NH, _DK = 12, 64
_N1 = _N2 = 64
_NJ = 12
_JB = 64
_KT = 8
_RB = 512
_LB = 1024

_NG = 8
_K1C = 8
_GW = (_N2 // _NG) * _D

_ar = np.arange(64)
_F64 = np.exp(-2j * np.pi * np.outer(_ar, _ar) / 64.0)
_TW = np.exp(-2j * np.pi * np.outer(_ar, _ar) / 4096.0)
_F_RE = np.ascontiguousarray(_F64.real.astype(np.float32))
_F_IM = np.ascontiguousarray(_F64.imag.astype(np.float32))
_TB3_RE = np.ascontiguousarray(np.repeat(_TW.real, _D, axis=1).astype(np.float32))
_TB3_IM = np.ascontiguousarray(np.repeat(_TW.imag, _D, axis=1).astype(np.float32))
_JONES8 = np.zeros((_GW, 64), dtype=np.float32)
_JONES8[np.arange(_GW), np.arange(_GW) // _D] = 1.0


def _proj_body(q_ref, k_ref, v_ref, wq_ref, wk_ref, wv_ref, wo_ref, b_ref,
               qo_ref, ko_ref, vo_ref):
    bq = b_ref[0, :]
    bk = b_ref[1, :]
    bv = b_ref[2, :]
    bo = b_ref[3, :]
    dot = lambda a, b: jnp.dot(a, b, preferred_element_type=jnp.float32)
    qo_ref[0] = dot(q_ref[0], wq_ref[...]) + bq[None, :]
    ko_ref[0] = dot(k_ref[0], wk_ref[...]) + bk[None, :]
    v = dot(v_ref[0], wv_ref[...]) + bv[None, :]
    vo = dot(v, wo_ref[...]) + bo[None, :]
    vo_ref[0, 0] = vo
    vo_ref[0, 1] = vo


def _f1_body(q_ref, k_ref, fre_ref, fim_ref, tre_ref, tim_ref,
             qcr_ref, qci_ref, kcr_ref, kci_ref):
    fre = fre_ref[...]
    fim = fim_ref[...]
    tre = tre_ref[...]
    tim = tim_ref[...]
    dot = lambda a, b: jnp.dot(a, b, preferred_element_type=jnp.float32)

    def stage1(x):
        br = dot(fre, x)
        bi = dot(fim, x)
        return br * tre - bi * tim, br * tim + bi * tre

    qcr, qci = stage1(q_ref[0])
    qcr_ref[0] = qcr
    qci_ref[0] = qci
    kcr, kci = stage1(k_ref[0])
    kcr_ref[0] = kcr
    kci_ref[0] = kci


def _f2_body(qcr_ref, qci_ref, kcr_ref, kci_ref, fre_ref, fim_ref,
             er_ref, ei_ref):
    fre = fre_ref[...]
    fim = fim_ref[...]
    dot = lambda a, b: jnp.dot(a, b, preferred_element_type=jnp.float32)

    def stage2(cr, ci):
        return dot(fre, cr) - dot(fim, ci), dot(fim, cr) + dot(fre, ci)

    for t in range(_K1C):
        qr, qi = stage2(qcr_ref[0, t], qci_ref[0, t])
        kr, ki = stage2(kcr_ref[0, t], kci_ref[0, t])
        sr = qr * kr + qi * ki
        si = qi * kr - qr * ki
        er_ref[0, t] = dot(fre, sr) + dot(fim, si)
        ei_ref[0, t] = dot(fre, si) - dot(fim, sr)


def _f3_body(er_ref, ei_ref, fre_ref, fim_ref, tre_ref, tim_ref, jones_ref,
             corr_ref, rp_ref):
    fre = fre_ref[...]
    fim = fim_ref[...]
    tre = tre_ref[...]
    tim = tim_ref[...]
    dot = lambda a, b: jnp.dot(a, b, preferred_element_type=jnp.float32)
    er = er_ref[0]
    ei = ei_ref[0]
    cr = er * tre + ei * tim
    ci = ei * tre - er * tim
    ar = (dot(fre, cr) + dot(fim, ci)) * (1.0 / 4096.0)
    corr_ref[0] = ar
    rp_ref[0] = dot(ar, jones_ref[...])


def _topk_body(r_ref, tau_ref, wgt_ref):
    rm = r_ref[0] + r_ref[1] + r_ref[2] + r_ref[3]
    row = jax.lax.broadcasted_iota(jnp.int32, (64, 64), 0)
    col = jax.lax.broadcasted_iota(jnp.int32, (64, 64), 1)
    lin = row * 64 + col
    big = jnp.int32(1 << 30)
    neg = jnp.float32(-3.0e38)
    tr = [[None] * _KT for _ in range(_B)]
    for i in range(_KT):
        m = jnp.max(rm)
        idx = jnp.min(jnp.where(rm == m, lin, big))
        tau_ref[i] = idx
        sel = lin == idx
        for b in range(_B):
            tr[b][i] = jnp.sum(jnp.where(sel, r_ref[b], 0.0)) * (1.0 / 768.0)
        rm = jnp.where(sel, neg, rm)
    for b in range(_B):
        mx = tr[b][0]
        for i in range(1, _KT):
            mx = jnp.maximum(mx, tr[b][i])
        es = [jnp.exp(tr[b][i] - mx) for i in range(_KT)]
        tot = es[0]
        for i in range(1, _KT):
            tot = tot + es[i]
        inv = 1.0 / tot
        for i in range(_KT):
            wgt_ref[b, i] = es[i] * inv


def _agg_body(tau_ref, wgt_ref, vo_ref, out_ref, *scratch):
    bufs = scratch[:_KT]
    sems = scratch[_KT]
    b = pl.program_id(0)
    l0 = pl.program_id(1) * _LB
    copies = []
    for i in range(_KT):
        start = pl.multiple_of((b * 2 * _L + l0 + tau_ref[i]) * _D, _D)
        cp = pltpu.make_async_copy(
            vo_ref.at[pl.ds(start, _LB * _D)], bufs[i], sems.at[i])
        cp.start()
        copies.append(cp)
    acc = None
    for i in range(_KT):
        copies[i].wait()
        term = wgt_ref[b, i] * bufs[i][...]
        acc = term if acc is None else acc + term
    out_ref[...] = acc


def kernel(Q, K, V, Wq, bq, Wk, bk, Wv, bv, Wo, bo):
    f32 = jnp.float32
    fre = jnp.asarray(_F_RE)
    fim = jnp.asarray(_F_IM)
    tre = jnp.asarray(_TB3_RE)
    tim = jnp.asarray(_TB3_IM)
    jones = jnp.asarray(_JONES8)

    wqt = Wq.T
    wkt = Wk.T
    wvt = Wv.T
    wot = Wo.T
    bias = jnp.stack([bq, bk, bv, bo])

    nrb = _L // _RB
    row_spec = pl.BlockSpec((1, _RB, _D), lambda b, i: (b, i, 0))
    w_spec = pl.BlockSpec((_D, _D), lambda b, i: (0, 0))
    b_spec = pl.BlockSpec((4, _D), lambda b, i: (0, 0))
    vo_spec = pl.BlockSpec((1, 2, _RB, _D), lambda b, i: (b, 0, i, 0))
    qf, kf, vop = pl.pallas_call(
        _proj_body,
        grid=(_B, nrb),
        in_specs=[row_spec, row_spec, row_spec, w_spec, w_spec, w_spec,
                  w_spec, b_spec],
        out_specs=[row_spec, row_spec, vo_spec],
        out_shape=[jax.ShapeDtypeStruct((_B, _L, _D), f32)] * 2
                  + [jax.ShapeDtypeStruct((_B, 2, _L, _D), f32)],
    )(Q, K, V, wqt, wkt, wvt, wot, bias)

    qn = qf.reshape(_B, _N1, _N2 * _D)
    kn = kf.reshape(_B, _N1, _N2 * _D)

    g_spec = pl.BlockSpec((1, 64, _GW), lambda g, b: (b, 0, g))
    c_spec = pl.BlockSpec((64, 64), lambda g, b: (0, 0))
    t_spec = pl.BlockSpec((64, _GW), lambda g, b: (0, g))
    fshape = jax.ShapeDtypeStruct((_B, _N1, _N2 * _D), f32)

    qcr, qci, kcr, kci = pl.pallas_call(
        _f1_body,
        grid=(_NG, _B),
        in_specs=[g_spec, g_spec, c_spec, c_spec, t_spec, t_spec],
        out_specs=[g_spec] * 4,
        out_shape=[fshape] * 4,
    )(qn, kn, fre, fim, tre, tim)

    rs = lambda x: x.reshape(_B, _N1, _N2, _D)
    k_spec = pl.BlockSpec((1, _K1C, 64, _D), lambda k1, b: (b, k1, 0, 0))
    c2_spec = pl.BlockSpec((64, 64), lambda k1, b: (0, 0))
    f2shape = jax.ShapeDtypeStruct((_B, _N1, _N2, _D), f32)
    er, ei = pl.pallas_call(
        _f2_body,
        grid=(_N1 // _K1C, _B),
        in_specs=[k_spec] * 4 + [c2_spec, c2_spec],
        out_specs=[k_spec] * 2,
        out_shape=[f2shape] * 2,
    )(rs(qcr), rs(qci), rs(kcr), rs(kci), fre, fim)

    jones_spec = pl.BlockSpec((_GW, 64), lambda g, b: (0, 0))
    rp_spec = pl.BlockSpec((1, 64, 64), lambda g, b: (b * _NG + g, 0, 0))
    corr_n, r_part = pl.pallas_call(
        _f3_body,
        grid=(_NG, _B),
        in_specs=[g_spec, g_spec, c_spec, c_spec, t_spec, t_spec,
                  jones_spec],
        out_specs=[g_spec, rp_spec],
        out_shape=[fshape, jax.ShapeDtypeStruct((_B * _NG, 64, 64), f32)],
    )(er.reshape(_B, _N1, _N2 * _D), ei.reshape(_B, _N1, _N2 * _D),
      fre, fim, tre, tim, jones)
    r_part = r_part.reshape(_B, _NG, 64, 64)

    corr_out = corr_n.reshape(_B, _L, _NH, _DK)
    r_out = (r_part[:, :, :, :_NG].transpose(0, 2, 1, 3)
             .reshape(_B, 64, 64))

    tau, wgt = pl.pallas_call(
        _topk_body,
        grid=(1,),
        in_specs=[pl.BlockSpec((_B, 64, 64), lambda i: (0, 0, 0))],
        out_specs=[pl.BlockSpec(memory_space=pltpu.SMEM),
                   pl.BlockSpec(memory_space=pltpu.SMEM)],
        out_shape=[jax.ShapeDtypeStruct((_KT,), jnp.int32),
                   jax.ShapeDtypeStruct((_B, _KT), f32)],
    )(r_out)

    vo_flat = vop.reshape(_B * 2 * _L * _D)

    nlb = _L // _LB
    out_flat = pl.pallas_call(
        _agg_body,
        grid=(_B, nlb),
        in_specs=[pl.BlockSpec(memory_space=pltpu.SMEM),
                  pl.BlockSpec(memory_space=pltpu.SMEM),
                  pl.BlockSpec(memory_space=pl.ANY)],
        out_specs=pl.BlockSpec((_LB * _D,), lambda b, i: (b * nlb + i,)),
        out_shape=jax.ShapeDtypeStruct((_B * _L * _D,), f32),
        scratch_shapes=[pltpu.VMEM((_LB * _D,), f32)] * _KT
                       + [pltpu.SemaphoreType.DMA((_KT,))],
    )(tau, wgt, vo_flat)

    return out_flat.reshape(_B, _L, _D), corr_out


# speedup vs baseline: 3.2481x; 1.0002x over previous
import numpy as np
import jax
import jax.numpy as jnp
from jax.experimental import pallas as pl
from jax.experimental.pallas import tpu as pltpu

_B, _L, _D = 4, 4096, 768
_NH, _DK = 12, 64
_N1 = _N2 = 64
_KT = 8
_RB = 512
_LB = 1024

_NG = 8
_K1C = 8
_GW = (_N2 // _NG) * _D

_ar = np.arange(64)
_F64 = np.exp(-2j * np.pi * np.outer(_ar, _ar) / 64.0)
_TW = np.exp(-2j * np.pi * np.outer(_ar, _ar) / 4096.0)
_F_RE = np.ascontiguousarray(_F64.real.astype(np.float32))
_F_IM = np.ascontiguousarray(_F64.imag.astype(np.float32))
_TB3_RE = np.ascontiguousarray(np.repeat(_TW.real, _D, axis=1).astype(np.float32))
_TB3_IM = np.ascontiguousarray(np.repeat(_TW.imag, _D, axis=1).astype(np.float32))
_JONES8 = np.zeros((_GW, 64), dtype=np.float32)
_JONES8[np.arange(_GW), np.arange(_GW) // _D] = 1.0


def _proj_body(q_ref, k_ref, v_ref, wq_ref, wk_ref, wv_ref, wo_ref, b_ref,
               qo_ref, ko_ref, vo_ref):
    bq = b_ref[0, :]
    bk = b_ref[1, :]
    bv = b_ref[2, :]
    bo = b_ref[3, :]
    dot = lambda a, b: jnp.dot(a, b, preferred_element_type=jnp.float32)
    qo_ref[0] = dot(q_ref[0], wq_ref[...]) + bq[None, :]
    ko_ref[0] = dot(k_ref[0], wk_ref[...]) + bk[None, :]
    v = dot(v_ref[0], wv_ref[...]) + bv[None, :]
    vo = dot(v, wo_ref[...]) + bo[None, :]
    vo_ref[0, 0] = vo
    vo_ref[0, 1] = vo


def _f1_body(q_ref, k_ref, fre_ref, fim_ref, tre_ref, tim_ref,
             qcr_ref, qci_ref, kcr_ref, kci_ref):
    fre = fre_ref[...]
    fim = fim_ref[...]
    tre = tre_ref[...]
    tim = tim_ref[...]
    dot = lambda a, b: jnp.dot(a, b, preferred_element_type=jnp.float32)

    def stage1(x):
        br = dot(fre, x)
        bi = dot(fim, x)
        return br * tre - bi * tim, br * tim + bi * tre

    qcr, qci = stage1(q_ref[0])
    qcr_ref[0] = qcr
    qci_ref[0] = qci
    kcr, kci = stage1(k_ref[0])
    kcr_ref[0] = kcr
    kci_ref[0] = kci


def _f2_body(qcr_ref, qci_ref, kcr_ref, kci_ref, fre_ref, fim_ref,
             er_ref, ei_ref):
    fre = fre_ref[...]
    fim = fim_ref[...]
    dot = lambda a, b: jnp.dot(a, b, preferred_element_type=jnp.float32)

    def stage2(cr, ci):
        return dot(fre, cr) - dot(fim, ci), dot(fim, cr) + dot(fre, ci)

    for t in range(_K1C):
        qr, qi = stage2(qcr_ref[0, t], qci_ref[0, t])
        kr, ki = stage2(kcr_ref[0, t], kci_ref[0, t])
        sr = qr * kr + qi * ki
        si = qi * kr - qr * ki
        er_ref[0, t] = dot(fre, sr) + dot(fim, si)
        ei_ref[0, t] = dot(fre, si) - dot(fim, sr)


def _f3_body(er_ref, ei_ref, fre_ref, fim_ref, tre_ref, tim_ref, jones_ref,
             corr_ref, rp_ref):
    fre = fre_ref[...]
    fim = fim_ref[...]
    tre = tre_ref[...]
    tim = tim_ref[...]
    dot = lambda a, b: jnp.dot(a, b, preferred_element_type=jnp.float32)
    er = er_ref[0]
    ei = ei_ref[0]
    cr = er * tre + ei * tim
    ci = ei * tre - er * tim
    ar = (dot(fre, cr) + dot(fim, ci)) * (1.0 / 4096.0)
    corr_ref[0] = ar
    rp_ref[0] = dot(ar, jones_ref[...])


def _topk_body(r_ref, tau_ref, wgt_ref):
    rm = r_ref[0] + r_ref[1] + r_ref[2] + r_ref[3]
    row = jax.lax.broadcasted_iota(jnp.int32, (64, 64), 0)
    col = jax.lax.broadcasted_iota(jnp.int32, (64, 64), 1)
    lin = row * 64 + col
    big = jnp.int32(1 << 30)
    neg = jnp.float32(-3.0e38)
    tr = [[None] * _KT for _ in range(_B)]
    for i in range(_KT):
        m = jnp.max(rm)
        idx = jnp.min(jnp.where(rm == m, lin, big))
        tau_ref[i] = idx
        sel = lin == idx
        for b in range(_B):
            tr[b][i] = jnp.sum(jnp.where(sel, r_ref[b], 0.0)) * (1.0 / 768.0)
        rm = jnp.where(sel, neg, rm)
    for b in range(_B):
        mx = tr[b][0]
        for i in range(1, _KT):
            mx = jnp.maximum(mx, tr[b][i])
        es = [jnp.exp(tr[b][i] - mx) for i in range(_KT)]
        tot = es[0]
        for i in range(1, _KT):
            tot = tot + es[i]
        inv = 1.0 / tot
        for i in range(_KT):
            wgt_ref[b, i] = es[i] * inv


def _agg_body(tau_ref, wgt_ref, vo_ref, out_ref, *scratch):
    bufs = scratch[:_KT]
    sems = scratch[_KT]
    b = pl.program_id(0)
    l0 = pl.program_id(1) * _LB
    copies = []
    for i in range(_KT):
        start = pl.multiple_of((b * 2 * _L + l0 + tau_ref[i]) * _D, _D)
        cp = pltpu.make_async_copy(
            vo_ref.at[pl.ds(start, _LB * _D)], bufs[i], sems.at[i])
        cp.start()
        copies.append(cp)
    acc = None
    for i in range(_KT):
        copies[i].wait()
        term = wgt_ref[b, i] * bufs[i][...]
        acc = term if acc is None else acc + term
    out_ref[...] = acc


def kernel(Q, K, V, Wq, bq, Wk, bk, Wv, bv, Wo, bo):
    f32 = jnp.float32
    fre = jnp.asarray(_F_RE)
    fim = jnp.asarray(_F_IM)
    tre = jnp.asarray(_TB3_RE)
    tim = jnp.asarray(_TB3_IM)
    jones = jnp.asarray(_JONES8)

    wqt = Wq.T
    wkt = Wk.T
    wvt = Wv.T
    wot = Wo.T
    bias = jnp.stack([bq, bk, bv, bo])

    nrb = _L // _RB
    row_spec = pl.BlockSpec((1, _RB, _D), lambda b, i: (b, i, 0))
    w_spec = pl.BlockSpec((_D, _D), lambda b, i: (0, 0))
    b_spec = pl.BlockSpec((4, _D), lambda b, i: (0, 0))
    vo_spec = pl.BlockSpec((1, 2, _RB, _D), lambda b, i: (b, 0, i, 0))
    qf, kf, vop = pl.pallas_call(
        _proj_body,
        grid=(_B, nrb),
        in_specs=[row_spec, row_spec, row_spec, w_spec, w_spec, w_spec,
                  w_spec, b_spec],
        out_specs=[row_spec, row_spec, vo_spec],
        out_shape=[jax.ShapeDtypeStruct((_B, _L, _D), f32)] * 2
                  + [jax.ShapeDtypeStruct((_B, 2, _L, _D), f32)],
    )(Q, K, V, wqt, wkt, wvt, wot, bias)

    qn = qf.reshape(_B, _N1, _N2 * _D)
    kn = kf.reshape(_B, _N1, _N2 * _D)

    g_spec = pl.BlockSpec((1, 64, _GW), lambda g, b: (b, 0, g))
    c_spec = pl.BlockSpec((64, 64), lambda g, b: (0, 0))
    t_spec = pl.BlockSpec((64, _GW), lambda g, b: (0, g))
    fshape = jax.ShapeDtypeStruct((_B, _N1, _N2 * _D), f32)

    qcr, qci, kcr, kci = pl.pallas_call(
        _f1_body,
        grid=(_NG, _B),
        in_specs=[g_spec, g_spec, c_spec, c_spec, t_spec, t_spec],
        out_specs=[g_spec] * 4,
        out_shape=[fshape] * 4,
    )(qn, kn, fre, fim, tre, tim)

    rs = lambda x: x.reshape(_B, _N1, _N2, _D)
    k_spec = pl.BlockSpec((1, _K1C, 64, _D), lambda k1, b: (b, k1, 0, 0))
    c2_spec = pl.BlockSpec((64, 64), lambda k1, b: (0, 0))
    f2shape = jax.ShapeDtypeStruct((_B, _N1, _N2, _D), f32)
    er, ei = pl.pallas_call(
        _f2_body,
        grid=(_N1 // _K1C, _B),
        in_specs=[k_spec] * 4 + [c2_spec, c2_spec],
        out_specs=[k_spec] * 2,
        out_shape=[f2shape] * 2,
    )(rs(qcr), rs(qci), rs(kcr), rs(kci), fre, fim)

    jones_spec = pl.BlockSpec((_GW, 64), lambda g, b: (0, 0))
    rp_spec = pl.BlockSpec((1, 64, 64), lambda g, b: (b * _NG + g, 0, 0))
    corr_n, r_part = pl.pallas_call(
        _f3_body,
        grid=(_NG, _B),
        in_specs=[g_spec, g_spec, c_spec, c_spec, t_spec, t_spec,
                  jones_spec],
        out_specs=[g_spec, rp_spec],
        out_shape=[fshape, jax.ShapeDtypeStruct((_B * _NG, 64, 64), f32)],
    )(er.reshape(_B, _N1, _N2 * _D), ei.reshape(_B, _N1, _N2 * _D),
      fre, fim, tre, tim, jones)
    r_part = r_part.reshape(_B, _NG, 64, 64)

    corr_out = corr_n.reshape(_B, _L, _NH, _DK)
    r_out = (r_part[:, :, :, :_NG].transpose(0, 2, 1, 3)
             .reshape(_B, 64, 64))

    tau, wgt = pl.pallas_call(
        _topk_body,
        grid=(1,),
        in_specs=[pl.BlockSpec((_B, 64, 64), lambda i: (0, 0, 0))],
        out_specs=[pl.BlockSpec(memory_space=pltpu.SMEM),
                   pl.BlockSpec(memory_space=pltpu.SMEM)],
        out_shape=[jax.ShapeDtypeStruct((_KT,), jnp.int32),
                   jax.ShapeDtypeStruct((_B, _KT), f32)],
    )(r_out)

    vo_flat = vop.reshape(_B * 2 * _L * _D)

    nlb = _L // _LB
    out_flat = pl.pallas_call(
        _agg_body,
        grid=(_B, nlb),
        in_specs=[pl.BlockSpec(memory_space=pltpu.SMEM),
                  pl.BlockSpec(memory_space=pltpu.SMEM),
                  pl.BlockSpec(memory_space=pl.ANY)],
        out_specs=pl.BlockSpec((_LB * _D,), lambda b, i: (b * nlb + i,)),
        out_shape=jax.ShapeDtypeStruct((_B * _L * _D,), f32),
        scratch_shapes=[pltpu.VMEM((_LB * _D,), f32)] * _KT
                       + [pltpu.SemaphoreType.DMA((_KT,))],
    )(tau, wgt, vo_flat)

    return out_flat.reshape(_B, _L, _D), corr_out
